# baseline (speedup 1.0000x reference)
import functools

import jax
import jax.numpy as jnp
from jax import lax
from jax.experimental import pallas as pl
from jax.experimental.pallas import tpu as pltpu
from jax.experimental.pallas import tpu_sc as plsc

L = 16
B = 4
N = 4096
NSLAB = 16
SLAB = N // NSLAB
SC_ = SLAB // L
NWB = 8
QS = N // NWB
QC = QS // L
HOFF = 3 * N
BOFF = 4 * N
STRIDE = 4 * N + NSLAB * L
INF = 3.0e38

_MESH = plsc.VectorSubcoreMesh(core_axis_name="c", subcore_axis_name="s")


def _treemax(v):
    t = [v[l] for l in range(L)]
    while len(t) > 1:
        t = [jnp.maximum(t[i], t[i + 1]) for i in range(0, len(t), 2)]
    return t[0]


def _treemin(v):
    t = [v[l] for l in range(L)]
    while len(t) > 1:
        t = [jnp.minimum(t[i], t[i + 1]) for i in range(0, len(t), 2)]
    return t[0]


def _nn_pass(qv, dv, k):
    def _qchunk(qc, acc):
        qo = (qc * NWB + k) * L
        xq = qv[pl.ds(qo, L)]
        yq = qv[pl.ds(N + qo, L)]
        zq = qv[pl.ds(2 * N + qo, L)]
        nq = qv[pl.ds(HOFF + qo, L)]
        aq = xq * -2.0
        bq = yq * -2.0
        cq = zq * -2.0
        xq_min = _treemin(xq)
        xq_max = _treemax(xq)
        yq_min = yq[0]
        yq_max = yq[L - 1]
        s = qo // SLAB
        base_s = s * SC_

        def _chunk(j, rm):
            do = j * L
            xd = dv[pl.ds(do, L)]
            yd = dv[pl.ds(N + do, L)]
            zd = dv[pl.ds(2 * N + do, L)]
            hc = dv[pl.ds(HOFF + do, L)]
            t = [hc[l] + aq * xd[l] + bq * yd[l] + cq * zd[l]
                 for l in range(L)]
            while len(t) > 1:
                t = [jnp.minimum(t[i], t[i + 1]) for i in range(0, len(t), 2)]
            return jnp.minimum(rm, t[0])

        @plsc.parallel_loop(base_s, base_s + SC_, unroll=4,
                            carry=jnp.full((L,), INF, jnp.float32))
        def rm(j, r):
            return _chunk(j, r)

        ub = _treemax(rm + nq)

        nl = jnp.zeros((), jnp.int32)
        nr = jnp.zeros((), jnp.int32)
        for t in range(NSLAB):
            bc = dv[pl.ds(BOFF + t * L, L)]
            xlo_t = bc[0]
            xhi_t = bc[1]
            gl = xq_min - xhi_t
            gr = xlo_t - xq_max
            nl = nl + jnp.where((gl > 0.0) & (gl * gl > ub), 1, 0)
            nr = nr + jnp.where((gr > 0.0) & (gr * gr > ub), 1, 0)

        @plsc.parallel_loop(nl * SC_, (NSLAB - nr) * SC_, unroll=4,
                            carry=rm)
        def rm2(j, r):
            return _chunk(j, r)

        return acc + rm2 + nq

    return lax.fori_loop(0, QC, _qchunk, jnp.zeros((L,), jnp.float32))


@functools.partial(
    pl.kernel,
    out_type=jax.ShapeDtypeStruct((2 * L * NWB * B,), jnp.float32),
    mesh=_MESH,
    scratch_types=[
        pltpu.VMEM((STRIDE,), jnp.float32),
        pltpu.VMEM((STRIDE,), jnp.float32),
        pltpu.VMEM((2 * L,), jnp.float32),
    ],
)
def _cd_kernel(p1_hbm, p2_hbm, out_hbm, p1v, p2v, obuf):
    cid = lax.axis_index("c")
    sid = lax.axis_index("s")
    b = cid * 2 + sid // NWB
    k = sid % NWB

    pltpu.sync_copy(p1_hbm.at[pl.ds(b * STRIDE, STRIDE)], p1v)
    pltpu.sync_copy(p2_hbm.at[pl.ds(b * STRIDE, STRIDE)], p2v)

    d1vec = _nn_pass(p1v, p2v, k)
    d2vec = _nn_pass(p2v, p1v, k)

    obuf[pl.ds(0, L)] = d1vec
    obuf[pl.ds(L, L)] = d2vec
    gwid = cid * 16 + sid
    pltpu.sync_copy(obuf, out_hbm.at[pl.ds(gwid * 2 * L, 2 * L)])


def _prep(p):
    ix = jnp.argsort(p[:, :, 0], axis=1)
    ps = jnp.take_along_axis(p, ix[:, :, None], axis=1)
    xs = ps[:, :, 0]
    xlo = xs[:, ::SLAB]
    xhi = xs[:, SLAB - 1::SLAB]
    psl = ps.reshape(B, NSLAB, SLAB, 3)
    iy = jnp.argsort(psl[:, :, :, 1], axis=2)
    psl = jnp.take_along_axis(psl, iy[:, :, :, None], axis=2)
    pp = psl.reshape(B, N, 3)
    coords = jnp.transpose(pp, (0, 2, 1)).reshape(B, 3 * N)
    norms = jnp.sum(pp * pp, axis=2)
    bounds = jnp.zeros((B, NSLAB, L), jnp.float32)
    bounds = bounds.at[:, :, 0].set(xlo).at[:, :, 1].set(xhi)
    packed = jnp.concatenate(
        [coords, norms, bounds.reshape(B, NSLAB * L)], axis=1)
    return packed.reshape(B * STRIDE)


def kernel(p1, p2):
    out = _cd_kernel(_prep(p1), _prep(p2))
    return jnp.sum(out) * (1.0 / (B * N))


# speedup vs baseline: 1.0247x; 1.0247x over previous
import functools

import jax
import jax.numpy as jnp
from jax import lax
from jax.experimental import pallas as pl
from jax.experimental.pallas import tpu as pltpu
from jax.experimental.pallas import tpu_sc as plsc

L = 16
B = 4
N = 4096
NSLAB = 16
SLAB = N // NSLAB
SC_ = SLAB // L
NWB = 8
QS = N // NWB
QC = QS // L
HOFF = 3 * N
BOFF = 4 * N
STRIDE = 4 * N + NSLAB * L
INF = 3.0e38

_MESH = plsc.VectorSubcoreMesh(core_axis_name="c", subcore_axis_name="s")


def _treemax(v):
    t = [v[l] for l in range(L)]
    while len(t) > 1:
        t = [jnp.maximum(t[i], t[i + 1]) for i in range(0, len(t), 2)]
    return t[0]


def _treemin(v):
    t = [v[l] for l in range(L)]
    while len(t) > 1:
        t = [jnp.minimum(t[i], t[i + 1]) for i in range(0, len(t), 2)]
    return t[0]


def _nn_pass(qv, dv, k):
    def _qchunk(qc, acc):
        qo = (qc * NWB + k) * L
        xq = qv[pl.ds(qo, L)]
        yq = qv[pl.ds(N + qo, L)]
        zq = qv[pl.ds(2 * N + qo, L)]
        nq = qv[pl.ds(HOFF + qo, L)]
        aq = xq * -2.0
        bq = yq * -2.0
        cq = zq * -2.0
        xq_min = _treemin(xq)
        xq_max = _treemax(xq)
        yq_min = yq[0]
        yq_max = yq[L - 1]
        s = qo // SLAB
        base_s = s * SC_

        def _chunk(j, rm):
            do = j * L
            xd = dv[pl.ds(do, L)]
            yd = dv[pl.ds(N + do, L)]
            zd = dv[pl.ds(2 * N + do, L)]
            hc = dv[pl.ds(HOFF + do, L)]
            t = [hc[l] + aq * xd[l] + bq * yd[l] + cq * zd[l]
                 for l in range(L)]
            while len(t) > 1:
                t = [jnp.minimum(t[i], t[i + 1]) for i in range(0, len(t), 2)]
            return jnp.minimum(rm, t[0])

        @plsc.parallel_loop(base_s, base_s + SC_, unroll=4,
                            carry=jnp.full((L,), INF, jnp.float32))
        def rm(j, r):
            return _chunk(j, r)

        ub = _treemax(rm + nq)

        nl = jnp.zeros((), jnp.int32)
        nr = jnp.zeros((), jnp.int32)
        for t in range(NSLAB):
            bc = dv[pl.ds(BOFF + t * L, L)]
            xlo_t = bc[0]
            xhi_t = bc[1]
            gl = xq_min - xhi_t
            gr = xlo_t - xq_max
            nl = nl + jnp.where((gl > 0.0) & (gl * gl > ub), 1, 0)
            nr = nr + jnp.where((gr > 0.0) & (gr * gr > ub), 1, 0)

        @plsc.parallel_loop(nl * SC_, base_s, unroll=4, carry=rm)
        def rm2(j, r):
            return _chunk(j, r)

        @plsc.parallel_loop(base_s + SC_, (NSLAB - nr) * SC_, unroll=4,
                            carry=rm2)
        def rm3(j, r):
            return _chunk(j, r)

        return acc + rm3 + nq

    return lax.fori_loop(0, QC, _qchunk, jnp.zeros((L,), jnp.float32))


@functools.partial(
    pl.kernel,
    out_type=jax.ShapeDtypeStruct((2 * L * NWB * B,), jnp.float32),
    mesh=_MESH,
    scratch_types=[
        pltpu.VMEM((STRIDE,), jnp.float32),
        pltpu.VMEM((STRIDE,), jnp.float32),
        pltpu.VMEM((2 * L,), jnp.float32),
    ],
)
def _cd_kernel(p1_hbm, p2_hbm, out_hbm, p1v, p2v, obuf):
    cid = lax.axis_index("c")
    sid = lax.axis_index("s")
    b = cid * 2 + sid // NWB
    k = sid % NWB

    pltpu.sync_copy(p1_hbm.at[pl.ds(b * STRIDE, STRIDE)], p1v)
    pltpu.sync_copy(p2_hbm.at[pl.ds(b * STRIDE, STRIDE)], p2v)

    d1vec = _nn_pass(p1v, p2v, k)
    d2vec = _nn_pass(p2v, p1v, k)

    obuf[pl.ds(0, L)] = d1vec
    obuf[pl.ds(L, L)] = d2vec
    gwid = cid * 16 + sid
    pltpu.sync_copy(obuf, out_hbm.at[pl.ds(gwid * 2 * L, 2 * L)])


def _prep(p):
    ix = jnp.argsort(p[:, :, 0], axis=1)
    ps = jnp.take_along_axis(p, ix[:, :, None], axis=1)
    xs = ps[:, :, 0]
    xlo = xs[:, ::SLAB]
    xhi = xs[:, SLAB - 1::SLAB]
    psl = ps.reshape(B, NSLAB, SLAB, 3)
    iy = jnp.argsort(psl[:, :, :, 1], axis=2)
    psl = jnp.take_along_axis(psl, iy[:, :, :, None], axis=2)
    pp = psl.reshape(B, N, 3)
    coords = jnp.transpose(pp, (0, 2, 1)).reshape(B, 3 * N)
    norms = jnp.sum(pp * pp, axis=2)
    bounds = jnp.zeros((B, NSLAB, L), jnp.float32)
    bounds = bounds.at[:, :, 0].set(xlo).at[:, :, 1].set(xhi)
    packed = jnp.concatenate(
        [coords, norms, bounds.reshape(B, NSLAB * L)], axis=1)
    return packed.reshape(B * STRIDE)


def kernel(p1, p2):
    out = _cd_kernel(_prep(p1), _prep(p2))
    return jnp.sum(out) * (1.0 / (B * N))


# speedup vs baseline: 1.0266x; 1.0019x over previous
import functools

import jax
import jax.numpy as jnp
from jax import lax
from jax.experimental import pallas as pl
from jax.experimental.pallas import tpu as pltpu
from jax.experimental.pallas import tpu_sc as plsc

L = 16
B = 4
N = 4096
NSLAB = 16
SLAB = N // NSLAB
SC_ = SLAB // L
NWB = 8
QS = N // NWB
QC = QS // L
HOFF = 3 * N
BOFF = 4 * N
STRIDE = 4 * N + NSLAB * L
INF = 3.0e38

_MESH = plsc.VectorSubcoreMesh(core_axis_name="c", subcore_axis_name="s")


def _treemax(v):
    t = [v[l] for l in range(L)]
    while len(t) > 1:
        t = [jnp.maximum(t[i], t[i + 1]) for i in range(0, len(t), 2)]
    return t[0]


def _treemin(v):
    t = [v[l] for l in range(L)]
    while len(t) > 1:
        t = [jnp.minimum(t[i], t[i + 1]) for i in range(0, len(t), 2)]
    return t[0]


def _nn_pass(qv, dv, k):
    def _qchunk(qc, acc):
        qo = (qc * NWB + k) * L
        xq = qv[pl.ds(qo, L)]
        yq = qv[pl.ds(N + qo, L)]
        zq = qv[pl.ds(2 * N + qo, L)]
        nq = qv[pl.ds(HOFF + qo, L)]
        aq = xq * -2.0
        bq = yq * -2.0
        cq = zq * -2.0
        xq_min = _treemin(xq)
        xq_max = _treemax(xq)
        yq_min = yq[0]
        yq_max = yq[L - 1]
        s = qo // SLAB
        base_s = s * SC_

        def _chunk(j, rm):
            do = j * L
            xd = dv[pl.ds(do, L)]
            yd = dv[pl.ds(N + do, L)]
            zd = dv[pl.ds(2 * N + do, L)]
            hc = dv[pl.ds(HOFF + do, L)]
            t = [hc[l] + aq * xd[l] + bq * yd[l] + cq * zd[l]
                 for l in range(L)]
            while len(t) > 1:
                t = [jnp.minimum(t[i], t[i + 1]) for i in range(0, len(t), 2)]
            return jnp.minimum(rm, t[0])

        @plsc.parallel_loop(base_s, base_s + SC_, unroll=8,
                            carry=jnp.full((L,), INF, jnp.float32))
        def rm(j, r):
            return _chunk(j, r)

        ub = _treemax(rm + nq)

        nl = jnp.zeros((), jnp.int32)
        nr = jnp.zeros((), jnp.int32)
        for t in range(NSLAB):
            bc = dv[pl.ds(BOFF + t * L, L)]
            xlo_t = bc[0]
            xhi_t = bc[1]
            gl = xq_min - xhi_t
            gr = xlo_t - xq_max
            nl = nl + jnp.where((gl > 0.0) & (gl * gl > ub), 1, 0)
            nr = nr + jnp.where((gr > 0.0) & (gr * gr > ub), 1, 0)

        @plsc.parallel_loop(nl * SC_, base_s, unroll=4, carry=rm)
        def rm2(j, r):
            return _chunk(j, r)

        @plsc.parallel_loop(base_s + SC_, (NSLAB - nr) * SC_, unroll=4,
                            carry=rm2)
        def rm3(j, r):
            return _chunk(j, r)

        return acc + rm3 + nq

    return lax.fori_loop(0, QC, _qchunk, jnp.zeros((L,), jnp.float32))


@functools.partial(
    pl.kernel,
    out_type=jax.ShapeDtypeStruct((2 * L * NWB * B,), jnp.float32),
    mesh=_MESH,
    scratch_types=[
        pltpu.VMEM((STRIDE,), jnp.float32),
        pltpu.VMEM((STRIDE,), jnp.float32),
        pltpu.VMEM((2 * L,), jnp.float32),
    ],
)
def _cd_kernel(p1_hbm, p2_hbm, out_hbm, p1v, p2v, obuf):
    cid = lax.axis_index("c")
    sid = lax.axis_index("s")
    b = cid * 2 + sid // NWB
    k = sid % NWB

    pltpu.sync_copy(p1_hbm.at[pl.ds(b * STRIDE, STRIDE)], p1v)
    pltpu.sync_copy(p2_hbm.at[pl.ds(b * STRIDE, STRIDE)], p2v)

    d1vec = _nn_pass(p1v, p2v, k)
    d2vec = _nn_pass(p2v, p1v, k)

    obuf[pl.ds(0, L)] = d1vec
    obuf[pl.ds(L, L)] = d2vec
    gwid = cid * 16 + sid
    pltpu.sync_copy(obuf, out_hbm.at[pl.ds(gwid * 2 * L, 2 * L)])


def _prep(p):
    ix = jnp.argsort(p[:, :, 0], axis=1)
    ps = jnp.take_along_axis(p, ix[:, :, None], axis=1)
    xs = ps[:, :, 0]
    xlo = xs[:, ::SLAB]
    xhi = xs[:, SLAB - 1::SLAB]
    psl = ps.reshape(B, NSLAB, SLAB, 3)
    iy = jnp.argsort(psl[:, :, :, 1], axis=2)
    psl = jnp.take_along_axis(psl, iy[:, :, :, None], axis=2)
    pp = psl.reshape(B, N, 3)
    coords = jnp.transpose(pp, (0, 2, 1)).reshape(B, 3 * N)
    norms = jnp.sum(pp * pp, axis=2)
    bounds = jnp.zeros((B, NSLAB, L), jnp.float32)
    bounds = bounds.at[:, :, 0].set(xlo).at[:, :, 1].set(xhi)
    packed = jnp.concatenate(
        [coords, norms, bounds.reshape(B, NSLAB * L)], axis=1)
    return packed.reshape(B * STRIDE)


def kernel(p1, p2):
    out = _cd_kernel(_prep(p1), _prep(p2))
    return jnp.sum(out) * (1.0 / (B * N))
